# Initial kernel scaffold; baseline (speedup 1.0000x reference)
import jax
import jax.numpy as jnp
from jax.experimental import pallas as pl


def kernel(quantized_f_embedding, edge_index, W1, b1, gamma, beta, W2, b2):
    raise NotImplementedError("write your pallas kernel here")


# speedup vs baseline: 18.3839x; 18.3839x over previous
import functools

import jax
import jax.numpy as jnp
from jax import lax
from jax.experimental import pallas as pl
from jax.experimental.pallas import tpu as pltpu
from jax.experimental.pallas import tpu_sc as plsc

N = 10000
D = 128
E = 320000
NC = 2
NS = 16
NW = NC * NS
EPW = E // NW
B = 80
NCHUNK = EPW // B
NP = 10240
RPT = NP // NS

_P = lax.Precision.HIGHEST

_mesh = plsc.VectorSubcoreMesh(
    core_axis_name="c", subcore_axis_name="s", num_cores=NC, num_subcores=NS)


@functools.partial(
    pl.kernel,
    out_type=jax.ShapeDtypeStruct((NC, NP, D), jnp.float32),
    mesh=_mesh,
    scratch_types=[
        pltpu.VMEM((NCHUNK, B), jnp.int32),
        pltpu.VMEM((B, D), jnp.float32),
        pltpu.VMEM_SHARED((NP, D), jnp.float32),
    ],
)
def _deg_sc(dst_hbm, out_hbm, dst_v, ones_v, acc_sh):
    c = lax.axis_index("c")
    s = lax.axis_index("s")
    wid = c * NS + s

    pltpu.sync_copy(dst_hbm.at[wid], dst_v)

    def zr(i, _):
        ones_v[i // 8, pl.ds((i % 8) * 16, 16)] = jnp.zeros((16,), jnp.float32)
        return 0

    lax.fori_loop(0, B * (D // 16), zr, 0)

    base = s * RPT
    for j in range(RPT // B):
        pltpu.sync_copy(ones_v, acc_sh.at[pl.ds(base + j * B, B)])

    def fl(i, _):
        ones_v[i // 8, pl.ds((i % 8) * 16, 16)] = jnp.full(
            (16,), 1.0, jnp.float32)
        return 0

    lax.fori_loop(0, B * (D // 16), fl, 0)
    plsc.subcore_barrier()

    def body(i, _):
        pltpu.sync_copy(ones_v, acc_sh.at[dst_v.at[i]], add=True)
        return 0

    lax.fori_loop(0, NCHUNK, body, 0)
    plsc.subcore_barrier()
    pltpu.sync_copy(acc_sh.at[pl.ds(base, RPT)],
                    out_hbm.at[c, pl.ds(base, RPT)])


@functools.partial(
    pl.kernel,
    out_type=jax.ShapeDtypeStruct((NC, NP, D), jnp.float32),
    mesh=_mesh,
    scratch_types=[
        pltpu.VMEM((NCHUNK, B), jnp.int32),
        pltpu.VMEM((NCHUNK, B), jnp.int32),
        pltpu.VMEM((B, D), jnp.float32),
        pltpu.VMEM_SHARED((NP, D), jnp.float32),
        pltpu.SemaphoreType.DMA,
    ],
)
def _seg_sc(h_hbm, src_hbm, dst_hbm, out_hbm,
            src_v, dst_v, rows_v, acc_sh, sem):
    c = lax.axis_index("c")
    s = lax.axis_index("s")
    wid = c * NS + s

    pltpu.sync_copy(src_hbm.at[wid], src_v)
    pltpu.sync_copy(dst_hbm.at[wid], dst_v)

    def zr(i, _):
        rows_v[i // 8, pl.ds((i % 8) * 16, 16)] = jnp.zeros((16,), jnp.float32)
        return 0

    lax.fori_loop(0, B * (D // 16), zr, 0)

    base = s * RPT
    for j in range(RPT // B):
        pltpu.sync_copy(rows_v, acc_sh.at[pl.ds(base + j * B, B)])
    rem = RPT % B
    if rem:
        pltpu.sync_copy(rows_v.at[pl.ds(0, rem)],
                        acc_sh.at[pl.ds(base + (RPT // B) * B, rem)])
    plsc.subcore_barrier()

    def body(i, _):
        pltpu.async_copy(h_hbm.at[src_v.at[i]], rows_v, sem).wait()
        pltpu.sync_copy(rows_v, acc_sh.at[dst_v.at[i]], add=True)
        return 0

    lax.fori_loop(0, NCHUNK, body, 0)
    plsc.subcore_barrier()
    pltpu.sync_copy(acc_sh.at[pl.ds(base, RPT)],
                    out_hbm.at[c, pl.ds(base, RPT)])


def _tc_a_body(x_ref, w1_ref, degp_ref, h_ref, dinv_ref):
    deg = degp_ref[0, 0:N, 0:1] + degp_ref[1, 0:N, 0:1] + 1.0
    dinv = lax.rsqrt(jnp.maximum(deg, 1e-12))
    h = jnp.dot(x_ref[...], w1_ref[...],
                preferred_element_type=jnp.float32, precision=_P)
    h_ref[...] = h * dinv
    dinv_ref[...] = dinv


def _tc_b_body(s1_ref, h1_ref, dinv_ref, b1_ref, g_ref, be_ref, w2_ref,
               h2_ref):
    dinv = dinv_ref[...]
    t = (s1_ref[0, 0:N] + s1_ref[1, 0:N] + h1_ref[...]) * dinv + b1_ref[...]
    mu = jnp.mean(t, axis=0, keepdims=True)
    var = jnp.mean((t - mu) * (t - mu), axis=0, keepdims=True)
    y = (t - mu) * lax.rsqrt(var + 1e-5) * g_ref[...] + be_ref[...]
    h2 = jnp.dot(y, w2_ref[...],
                 preferred_element_type=jnp.float32, precision=_P)
    h2_ref[...] = h2 * dinv


def _tc_c_body(s2_ref, h2_ref, dinv_ref, b2_ref, out_ref):
    out_ref[...] = ((s2_ref[0, 0:N] + s2_ref[1, 0:N] + h2_ref[...]) * dinv_ref[...]
                    + b2_ref[...])


_tc_a = pl.pallas_call(
    _tc_a_body,
    out_shape=[jax.ShapeDtypeStruct((N, D), jnp.float32),
               jax.ShapeDtypeStruct((N, 1), jnp.float32)],
)

_tc_b = pl.pallas_call(
    _tc_b_body,
    out_shape=jax.ShapeDtypeStruct((N, D), jnp.float32),
)

_tc_c = pl.pallas_call(
    _tc_c_body,
    out_shape=jax.ShapeDtypeStruct((N, D), jnp.float32),
)


def kernel(quantized_f_embedding, edge_index, W1, b1, gamma, beta, W2, b2):
    x = quantized_f_embedding
    src3 = edge_index[0].reshape(NW, NCHUNK, B)
    dst3 = edge_index[1].reshape(NW, NCHUNK, B)
    b1r = b1.reshape(1, D)
    b2r = b2.reshape(1, D)
    gr = gamma.reshape(1, D)
    ber = beta.reshape(1, D)

    degp = _deg_sc(dst3)
    h1p, dinv = _tc_a(x, W1, degp)
    s1p = _seg_sc(h1p, src3, dst3)
    h2p = _tc_b(s1p, h1p, dinv, b1r, gr, ber, W2)
    s2p = _seg_sc(h2p, src3, dst3)
    out = _tc_c(s2p, h2p, dinv, b2r)
    return out
